# baseline (speedup 1.0000x reference)
import functools

import jax
import jax.numpy as jnp
from jax import lax
from jax.experimental import pallas as pl
from jax.experimental.pallas import tpu as pltpu
from jax.experimental.pallas import tpu_sc as plsc

NC = 2
NS = 16
NW = NC * NS
CH = 128


def _mesh():
    return plsc.VectorSubcoreMesh(core_axis_name="c", subcore_axis_name="s")


def _make_deg(npad, ew):
    @functools.partial(
        pl.kernel,
        mesh=_mesh(),
        out_type=jax.ShapeDtypeStruct((NW * npad,), jnp.float32),
        scratch_types=[
            pltpu.VMEM((npad,), jnp.float32),
            pltpu.VMEM((ew,), jnp.int32),
        ],
        compiler_params=pltpu.CompilerParams(
            use_tc_tiling_on_sc=False, needs_layout_passes=False),
    )
    def deg_kernel(dst_hbm, out_hbm, deg_v, didx):
        c = lax.axis_index("c")
        s = lax.axis_index("s")
        wid = s * NC + c

        def zero(i, carry):
            deg_v[pl.ds(i * 16, 16)] = jnp.zeros((16,), jnp.float32)
            return carry

        lax.fori_loop(0, npad // 16, zero, 0)
        pltpu.sync_copy(dst_hbm.at[wid], didx)
        ones = jnp.ones((16,), jnp.float32)

        def body(i, carry):
            idx = didx[pl.ds(i * 16, 16)]
            plsc.addupdate_scatter(deg_v, [idx], ones)
            return carry

        lax.fori_loop(0, ew // 16, body, 0)
        pltpu.sync_copy(deg_v, out_hbm.at[pl.ds(wid * npad, npad)])

    return deg_kernel


def _make_prop(npad, nch, f, nbuf, kblk, stage):
    assert nch % (nbuf * kblk) == 0
    ngrp = nch // (nbuf * kblk)

    @functools.partial(
        pl.kernel,
        mesh=_mesh(),
        out_type=jax.ShapeDtypeStruct((NC, npad, f), jnp.float32),
        scratch_types=[
            pltpu.VMEM_SHARED((npad, f), jnp.float32),
            pltpu.VMEM_SHARED((npad if stage else 8, f), jnp.float32),
            pltpu.VMEM((nch // kblk, kblk * CH), jnp.int32),
            pltpu.VMEM((nch // kblk, kblk * CH), jnp.int32),
            pltpu.VMEM((nbuf, kblk * CH, f), jnp.float32),
            pltpu.VMEM((npad // NS, f), jnp.float32),
            [pltpu.SemaphoreType.DMA] * nbuf,
            [pltpu.SemaphoreType.DMA] * nbuf,
        ],
        compiler_params=pltpu.CompilerParams(use_tc_tiling_on_sc=False),
    )
    def prop_kernel(h_hbm, src_hbm, dst_hbm, zeros_hbm, out_hbm,
                    acc_sh, h_sh, sidx, didx, rows, bounce, gsem, ssem):
        c = lax.axis_index("c")
        s = lax.axis_index("s")
        wid = s * NC + c
        stripe = npad // NS
        sl = pl.ds(s * stripe, stripe)
        pltpu.sync_copy(zeros_hbm.at[sl], bounce)
        pltpu.sync_copy(bounce, acc_sh.at[sl])
        if stage:
            pltpu.sync_copy(h_hbm.at[sl], bounce)
            pltpu.sync_copy(bounce, h_sh.at[sl])
        pltpu.sync_copy(src_hbm.at[wid], sidx)
        pltpu.sync_copy(dst_hbm.at[wid], didx)
        plsc.subcore_barrier()
        gsrc = h_sh if stage else h_hbm

        def gather(q, b):
            pltpu.async_copy(gsrc.at[sidx.at[q]], rows.at[b], gsem[b])

        def gather_wait(q, b):
            pltpu.make_async_copy(gsrc.at[sidx.at[q]], rows.at[b], gsem[b]).wait()

        def scatter(q, b):
            pltpu.async_copy(rows.at[b], acc_sh.at[didx.at[q]], ssem[b], add=True)

        def scatter_wait(q, b):
            pltpu.make_async_copy(rows.at[b], acc_sh.at[didx.at[q]], ssem[b]).wait()

        for b in range(nbuf):
            gather(b, b)

        def body(g, carry):
            for b in range(nbuf):
                q = g * nbuf + b
                gather_wait(q, b)
                scatter(q, b)
            for b in range(nbuf):
                q = g * nbuf + b
                scatter_wait(q, b)

                @pl.when(g + 1 < ngrp)
                def _():
                    gather(q + nbuf, b)

            return carry

        lax.fori_loop(0, ngrp, body, 0)
        plsc.subcore_barrier()
        pltpu.sync_copy(acc_sh.at[sl], bounce)
        pltpu.sync_copy(bounce, out_hbm.at[c, sl])

    return prop_kernel


def _tc_first(deg_t, x, w1):
    npad = x.shape[0]
    h1dim = w1.shape[1]

    def body(deg_ref, x_ref, w_ref, h_ref, dinv_ref):
        deg = jnp.sum(deg_ref[...], axis=1, keepdims=True) + 1.0
        dinv = lax.rsqrt(deg)
        h = jnp.dot(x_ref[...], w_ref[...], preferred_element_type=jnp.float32)
        h_ref[...] = h * dinv
        dinv_ref[...] = dinv

    return pl.pallas_call(
        body,
        out_shape=(
            jax.ShapeDtypeStruct((npad, h1dim), jnp.float32),
            jax.ShapeDtypeStruct((npad, 1), jnp.float32),
        ),
    )(deg_t, x, w1)


def _tc_mid(a0, a1, h1s, dinv, w2p):
    npad = a0.shape[0]
    f2 = w2p.shape[1]

    def body(a0_ref, a1_ref, h_ref, dinv_ref, w_ref, out_ref):
        t = (a0_ref[...] + a1_ref[...] + h_ref[...]) * dinv_ref[...]
        g = jnp.maximum(t, 0.0)
        h2 = jnp.dot(g, w_ref[...], preferred_element_type=jnp.float32)
        out_ref[...] = h2 * dinv_ref[...]

    return pl.pallas_call(
        body,
        out_shape=jax.ShapeDtypeStruct((npad, f2), jnp.float32),
    )(a0, a1, h1s, dinv, w2p)


def _tc_final(a0, a1, h2s, dinv):
    npad, f2 = a0.shape

    def body(a0_ref, a1_ref, h_ref, dinv_ref, out_ref):
        out_ref[...] = (a0_ref[...] + a1_ref[...] + h_ref[...]) * dinv_ref[...]

    return pl.pallas_call(
        body,
        out_shape=jax.ShapeDtypeStruct((npad, f2), jnp.float32),
    )(a0, a1, h2s, dinv)


def kernel(node_features, edge_features, latent_features, edge_index, device, W1, W2):
    x = node_features
    n = x.shape[0]
    e = edge_index.shape[1]
    h1dim = W1.shape[1]
    h2dim = W2.shape[1]
    f2 = 16

    nbuf = 2
    kblk = 8
    npad = ((n + 1 + 127) // 128) * 128
    nch = (e + NW * CH - 1) // (NW * CH)
    nch = ((nch + nbuf * kblk - 1) // (nbuf * kblk)) * (nbuf * kblk)
    etot = NW * nch * CH

    src = edge_index[0]
    dst = edge_index[1]
    src_p = jnp.concatenate(
        [src, jnp.zeros((etot - e,), jnp.int32)]).reshape(NW, nch, CH)
    dst_p = jnp.concatenate(
        [dst, jnp.full((etot - e,), n, jnp.int32)]).reshape(NW, nch, CH)
    x_p = jnp.pad(x, ((0, npad - n), (0, 0)))
    w2p = jnp.pad(W2, ((0, 0), (0, f2 - h2dim)))
    zeros_f1 = jnp.zeros((npad, h1dim), jnp.float32)
    zeros_f2 = jnp.zeros((npad, f2), jnp.float32)

    dst_flat = dst_p.reshape(NW, nch * CH)
    deg_parts = _make_deg(npad, nch * CH)(dst_flat).reshape(NW, npad)
    h1s, dinv = _tc_first(deg_parts.T, x_p, W1)
    src_b = src_p.reshape(NW, nch // kblk, kblk * CH)
    dst_b = dst_p.reshape(NW, nch // kblk, kblk * CH)
    acc1 = _make_prop(npad, nch, h1dim, nbuf, kblk, False)(h1s, src_b, dst_b, zeros_f1)
    h2s = _tc_mid(acc1[0], acc1[1], h1s, dinv, w2p)
    acc2 = _make_prop(npad, nch, f2, nbuf, kblk, False)(h2s, src_b, dst_b, zeros_f2)
    outp = _tc_final(acc2[0], acc2[1], h2s, dinv)
    return outp[:n, :h2dim]


# speedup vs baseline: 1.6289x; 1.6289x over previous
import functools

import jax
import jax.numpy as jnp
from jax import lax
from jax.experimental import pallas as pl
from jax.experimental.pallas import tpu as pltpu
from jax.experimental.pallas import tpu_sc as plsc

NC = 2
NS = 16
NW = NC * NS
CH = 128


def _mesh():
    return plsc.VectorSubcoreMesh(core_axis_name="c", subcore_axis_name="s")


def _make_deg(npad, ew):
    @functools.partial(
        pl.kernel,
        mesh=_mesh(),
        out_type=jax.ShapeDtypeStruct((NW * npad,), jnp.float32),
        scratch_types=[
            pltpu.VMEM((npad,), jnp.float32),
            pltpu.VMEM((ew,), jnp.int32),
        ],
        compiler_params=pltpu.CompilerParams(
            use_tc_tiling_on_sc=False, needs_layout_passes=False),
    )
    def deg_kernel(dst_hbm, out_hbm, deg_v, didx):
        c = lax.axis_index("c")
        s = lax.axis_index("s")
        wid = s * NC + c

        def zero(i, carry):
            deg_v[pl.ds(i * 16, 16)] = jnp.zeros((16,), jnp.float32)
            return carry

        lax.fori_loop(0, npad // 16, zero, 0)
        pltpu.sync_copy(dst_hbm.at[wid], didx)
        ones = jnp.ones((16,), jnp.float32)

        def body(i, carry):
            idx = didx[pl.ds(i * 16, 16)]
            plsc.addupdate_scatter(deg_v, [idx], ones)
            return carry

        lax.fori_loop(0, ew // 16, body, 0)
        pltpu.sync_copy(deg_v, out_hbm.at[pl.ds(wid * npad, npad)])

    return deg_kernel


def _make_prop(npad, nch, f, nbuf, kblk, stage):
    assert nch % (nbuf * kblk) == 0
    ngrp = nch // (nbuf * kblk)

    @functools.partial(
        pl.kernel,
        mesh=_mesh(),
        out_type=jax.ShapeDtypeStruct((NC, npad, f), jnp.float32),
        scratch_types=[
            pltpu.VMEM_SHARED((npad, f), jnp.float32),
            pltpu.VMEM_SHARED((npad if stage else 8, f), jnp.float32),
            pltpu.VMEM((nch // kblk, kblk * CH), jnp.int32),
            pltpu.VMEM((nch // kblk, kblk * CH), jnp.int32),
            pltpu.VMEM((nbuf, kblk * CH, f), jnp.float32),
            pltpu.VMEM((npad // NS, f), jnp.float32),
            [pltpu.SemaphoreType.DMA] * nbuf,
            [pltpu.SemaphoreType.DMA] * nbuf,
        ],
        compiler_params=pltpu.CompilerParams(use_tc_tiling_on_sc=False),
    )
    def prop_kernel(h_hbm, src_hbm, dst_hbm, out_hbm,
                    acc_sh, h_sh, sidx, didx, rows, bounce, gsem, ssem):
        c = lax.axis_index("c")
        s = lax.axis_index("s")
        wid = s * NC + c
        stripe = npad // NS
        sl = pl.ds(s * stripe, stripe)
        pltpu.sync_copy(h_hbm.at[sl], bounce)
        pltpu.sync_copy(bounce, acc_sh.at[sl])
        if stage:
            pltpu.sync_copy(bounce, h_sh.at[sl])
        pltpu.sync_copy(src_hbm.at[wid], sidx)
        pltpu.sync_copy(dst_hbm.at[wid], didx)
        plsc.subcore_barrier()
        gsrc = h_sh if stage else h_hbm

        def gather(q, b):
            pltpu.async_copy(gsrc.at[sidx.at[q]], rows.at[b], gsem[b])

        def gather_wait(q, b):
            pltpu.make_async_copy(gsrc.at[sidx.at[q]], rows.at[b], gsem[b]).wait()

        def scatter(q, b):
            pltpu.async_copy(rows.at[b], acc_sh.at[didx.at[q]], ssem[b], add=True)

        def scatter_wait(q, b):
            pltpu.make_async_copy(rows.at[b], acc_sh.at[didx.at[q]], ssem[b]).wait()

        for b in range(nbuf):
            gather(b, b)

        def body(g, carry):
            for b in range(nbuf):
                q = g * nbuf + b
                gather_wait(q, b)
                scatter(q, b)
            for b in range(nbuf):
                q = g * nbuf + b
                scatter_wait(q, b)

                @pl.when(g + 1 < ngrp)
                def _():
                    gather(q + nbuf, b)

            return carry

        lax.fori_loop(0, ngrp, body, 0)
        plsc.subcore_barrier()
        pltpu.sync_copy(acc_sh.at[sl], bounce)
        pltpu.sync_copy(bounce, out_hbm.at[c, sl])

    return prop_kernel


def _make_prop_cols(npad, nq, fh, nbuf, chunk):
    assert nq % nbuf == 0
    ngrp = nq // nbuf

    @functools.partial(
        pl.kernel,
        mesh=_mesh(),
        out_type=jax.ShapeDtypeStruct((NC, npad, fh), jnp.float32),
        scratch_types=[
            pltpu.VMEM_SHARED((npad, fh), jnp.float32),
            pltpu.VMEM_SHARED((npad, fh), jnp.float32),
            pltpu.VMEM((nq, chunk), jnp.int32),
            pltpu.VMEM((nq, chunk), jnp.int32),
            pltpu.VMEM((nbuf, chunk, fh), jnp.float32),
            pltpu.VMEM((npad // NS, fh), jnp.float32),
            [pltpu.SemaphoreType.DMA] * nbuf,
            [pltpu.SemaphoreType.DMA] * nbuf,
        ],
        compiler_params=pltpu.CompilerParams(use_tc_tiling_on_sc=False),
    )
    def prop_kernel(hl_hbm, hr_hbm, src_hbm, dst_hbm, out_hbm,
                    acc_sh, h_sh, sidx, didx, rows, bounce, gsem, ssem):
        c = lax.axis_index("c")
        s = lax.axis_index("s")
        stripe = npad // NS
        sl = pl.ds(s * stripe, stripe)

        @pl.when(c == 0)
        def _():
            pltpu.sync_copy(hl_hbm.at[sl], bounce)

        @pl.when(c == 1)
        def _():
            pltpu.sync_copy(hr_hbm.at[sl], bounce)

        pltpu.sync_copy(bounce, acc_sh.at[sl])
        pltpu.sync_copy(bounce, h_sh.at[sl])
        pltpu.sync_copy(src_hbm.at[s], sidx)
        pltpu.sync_copy(dst_hbm.at[s], didx)
        plsc.subcore_barrier()

        def gather(q, b):
            pltpu.async_copy(h_sh.at[sidx.at[q]], rows.at[b], gsem[b])

        def gather_wait(q, b):
            pltpu.make_async_copy(h_sh.at[sidx.at[q]], rows.at[b], gsem[b]).wait()

        def scatter(q, b):
            pltpu.async_copy(rows.at[b], acc_sh.at[didx.at[q]], ssem[b], add=True)

        def scatter_wait(q, b):
            pltpu.make_async_copy(rows.at[b], acc_sh.at[didx.at[q]], ssem[b]).wait()

        for b in range(nbuf):
            gather(b, b)

        def body(g, carry):
            for b in range(nbuf):
                q = g * nbuf + b
                gather_wait(q, b)
                scatter(q, b)
            for b in range(nbuf):
                q = g * nbuf + b
                scatter_wait(q, b)

                @pl.when(g + 1 < ngrp)
                def _():
                    gather(q + nbuf, b)

            return carry

        lax.fori_loop(0, ngrp, body, 0)
        plsc.subcore_barrier()
        pltpu.sync_copy(acc_sh.at[sl], bounce)
        pltpu.sync_copy(bounce, out_hbm.at[c, sl])

    return prop_kernel


def _tc_first(deg_t, x, w1):
    npad = x.shape[0]
    h1dim = w1.shape[1]
    fh = h1dim // 2

    def body(deg_ref, x_ref, w_ref, hl_ref, hr_ref, dinv_ref):
        deg = jnp.sum(deg_ref[...], axis=1, keepdims=True) + 1.0
        dinv = lax.rsqrt(deg)
        h = jnp.dot(x_ref[...], w_ref[...], preferred_element_type=jnp.float32)
        hs = h * dinv
        hl_ref[...] = hs[:, :fh]
        hr_ref[...] = hs[:, fh:]
        dinv_ref[...] = dinv

    return pl.pallas_call(
        body,
        out_shape=(
            jax.ShapeDtypeStruct((npad, fh), jnp.float32),
            jax.ShapeDtypeStruct((npad, fh), jnp.float32),
            jax.ShapeDtypeStruct((npad, 1), jnp.float32),
        ),
    )(deg_t, x, w1)


def _tc_mid(a0, a1, dinv, w2p):
    npad = a0.shape[0]
    f2 = w2p.shape[1]

    def body(a0_ref, a1_ref, dinv_ref, w_ref, out_ref):
        t = jnp.concatenate([a0_ref[...], a1_ref[...]], axis=1) * dinv_ref[...]
        g = jnp.maximum(t, 0.0)
        h2 = jnp.dot(g, w_ref[...], preferred_element_type=jnp.float32)
        out_ref[...] = h2 * dinv_ref[...]

    return pl.pallas_call(
        body,
        out_shape=jax.ShapeDtypeStruct((npad, f2), jnp.float32),
    )(a0, a1, dinv, w2p)


def _tc_final(a0, a1, h2s, dinv):
    npad, f2 = a0.shape

    def body(a0_ref, a1_ref, h_ref, dinv_ref, out_ref):
        out_ref[...] = (a0_ref[...] + a1_ref[...] - h_ref[...]) * dinv_ref[...]

    return pl.pallas_call(
        body,
        out_shape=jax.ShapeDtypeStruct((npad, f2), jnp.float32),
    )(a0, a1, h2s, dinv)


def kernel(node_features, edge_features, latent_features, edge_index, device, W1, W2):
    x = node_features
    n = x.shape[0]
    e = edge_index.shape[1]
    h1dim = W1.shape[1]
    h2dim = W2.shape[1]
    f2 = 16

    nbuf = 2
    kblk = 8
    npad = ((n + 1 + 127) // 128) * 128
    nch = (e + NW * CH - 1) // (NW * CH)
    nch = ((nch + nbuf * kblk - 1) // (nbuf * kblk)) * (nbuf * kblk)
    etot = NW * nch * CH

    src = edge_index[0]
    dst = edge_index[1]
    src_p = jnp.concatenate(
        [src, jnp.zeros((etot - e,), jnp.int32)]).reshape(NW, nch, CH)
    dst_p = jnp.concatenate(
        [dst, jnp.full((etot - e,), n, jnp.int32)]).reshape(NW, nch, CH)
    x_p = jnp.pad(x, ((0, npad - n), (0, 0)))
    w2p = jnp.pad(W2, ((0, 0), (0, f2 - h2dim)))

    chunk = kblk * CH
    dst_flat = dst_p.reshape(NW, nch * CH)
    deg_parts = _make_deg(npad, nch * CH)(dst_flat).reshape(NW, npad)
    hl, hr, dinv = _tc_first(deg_parts.T, x_p, W1)
    nq1 = etot // (NS * chunk)
    src_q = src_p.reshape(NS, nq1, chunk)
    dst_q = dst_p.reshape(NS, nq1, chunk)
    acc1 = _make_prop_cols(npad, nq1, h1dim // 2, nbuf, chunk)(hl, hr, src_q, dst_q)
    h2s = _tc_mid(acc1[0], acc1[1], dinv, w2p)
    src_b = src_p.reshape(NW, nch // kblk, chunk)
    dst_b = dst_p.reshape(NW, nch // kblk, chunk)
    acc2 = _make_prop(npad, nch, f2, nbuf, kblk, True)(h2s, src_b, dst_b)
    outp = _tc_final(acc2[0], acc2[1], h2s, dinv)
    return outp[:n, :h2dim]
